# baseline (speedup 1.0000x reference)
import jax
import jax.numpy as jnp
from jax import lax
from jax.experimental import pallas as pl
from jax.experimental.pallas import tpu as pltpu
from jax.experimental.pallas import tpu_sc as plsc

_B, _L, _D = 4, 8192, 1024
_NC, _NS = 2, 16
_NW = _NC * _NS
_ROWS_W = _L // _NW
_CH = 16
_NCHUNK = _ROWS_W // _CH
_CHW = _CH * _D


def _sc_body(in_hbm, pos_hbm, out_hbm, a_v, p_v):
    wid = lax.axis_index("s") * _NC + lax.axis_index("c")
    l0w = wid * _ROWS_W * _D

    def chunk(c, carry):
        pos_off = l0w + c * _CHW
        pltpu.sync_copy(pos_hbm.at[pl.ds(pos_off, _CHW)], p_v)

        def per_batch(b, carry2):
            in_off = b * (_L * _D) + pos_off
            pltpu.sync_copy(in_hbm.at[pl.ds(in_off, _CHW)], a_v)

            @plsc.parallel_loop(0, _CHW, 16, unroll=8)
            def _add(k):
                a_v[pl.ds(k, 16)] = a_v[pl.ds(k, 16)] + p_v[pl.ds(k, 16)]

            pltpu.sync_copy(a_v, out_hbm.at[pl.ds(in_off, _CHW)])
            return carry2

        lax.fori_loop(0, _B, per_batch, 0)
        return carry

    lax.fori_loop(0, _NCHUNK, chunk, 0)


def kernel(inputs, pos_table):
    B, L, D = inputs.shape
    mesh = plsc.VectorSubcoreMesh(core_axis_name="c", subcore_axis_name="s")
    out = pl.kernel(
        _sc_body,
        out_type=jax.ShapeDtypeStruct((B * L * D,), jnp.float32),
        mesh=mesh,
        scratch_types=[
            pltpu.VMEM((_CHW,), jnp.float32),
            pltpu.VMEM((_CHW,), jnp.float32),
        ],
    )(inputs.reshape(-1), pos_table.reshape(-1))
    return out.reshape(B, L, D)


# speedup vs baseline: 2.3582x; 2.3582x over previous
import jax
import jax.numpy as jnp
from jax import lax
from jax.experimental import pallas as pl
from jax.experimental.pallas import tpu as pltpu
from jax.experimental.pallas import tpu_sc as plsc

_B, _L, _D = 4, 8192, 1024
_NC, _NS = 2, 16
_NW = _NC * _NS
_ROWS_W = _L // _NW
_CH = 16
_NCHUNK = _ROWS_W // _CH


def _sc_body(in_hbm, pos_hbm, out_hbm, a0, a1, a2, a3, p_v):
    wid = lax.axis_index("s") * _NC + lax.axis_index("c")
    l0w = wid * _ROWS_W

    def chunk(c, carry):
        row0 = l0w + c * _CH
        pltpu.sync_copy(pos_hbm.at[pl.ds(row0, _CH)], p_v)
        pltpu.sync_copy(in_hbm.at[0, pl.ds(row0, _CH)], a0)
        pltpu.sync_copy(in_hbm.at[1, pl.ds(row0, _CH)], a1)
        pltpu.sync_copy(in_hbm.at[2, pl.ds(row0, _CH)], a2)
        pltpu.sync_copy(in_hbm.at[3, pl.ds(row0, _CH)], a3)

        def row(i, carry2):
            @plsc.parallel_loop(0, _D, 16, unroll=8)
            def _vec(j):
                pv = p_v[i, pl.ds(j, 16)]
                plsc.addupdate(a0.at[i, pl.ds(j, 16)], pv)
                plsc.addupdate(a1.at[i, pl.ds(j, 16)], pv)
                plsc.addupdate(a2.at[i, pl.ds(j, 16)], pv)
                plsc.addupdate(a3.at[i, pl.ds(j, 16)], pv)

            return carry2

        lax.fori_loop(0, _CH, row, 0)

        pltpu.sync_copy(a0, out_hbm.at[0, pl.ds(row0, _CH)])
        pltpu.sync_copy(a1, out_hbm.at[1, pl.ds(row0, _CH)])
        pltpu.sync_copy(a2, out_hbm.at[2, pl.ds(row0, _CH)])
        pltpu.sync_copy(a3, out_hbm.at[3, pl.ds(row0, _CH)])
        return carry

    lax.fori_loop(0, _NCHUNK, chunk, 0)


def kernel(inputs, pos_table):
    B, L, D = inputs.shape
    mesh = plsc.VectorSubcoreMesh(core_axis_name="c", subcore_axis_name="s")
    buf = lambda: pltpu.VMEM((_CH, _D), jnp.float32)
    return pl.kernel(
        _sc_body,
        out_type=jax.ShapeDtypeStruct((B, L, D), jnp.float32),
        mesh=mesh,
        scratch_types=[buf(), buf(), buf(), buf(), buf()],
        compiler_params=pltpu.CompilerParams(use_tc_tiling_on_sc=True),
    )(inputs, pos_table)
